# baseline (speedup 1.0000x reference)
import functools

import jax
import jax.numpy as jnp
from jax import lax
from jax.experimental import pallas as pl
from jax.experimental.pallas import tpu as pltpu
from jax.experimental.pallas import tpu_sc as plsc

H = 512
W = 512
B = 4
NF = 100000
NTRI = B * NF
NCH = 16
CHUNK = 128
ROWS = B * H
LANES = 16

_HASH = 2654435761 - (1 << 32)

_SPLAT_DNUMS = lax.GatherDimensionNumbers(
    offset_dims=(), collapsed_slice_dims=(0,), start_index_map=(0,))


def _splat(vec, j):
  idx = jnp.full((LANES, 1), j, dtype=jnp.int32)
  return lax.gather(vec, idx, _SPLAT_DNUMS, (1,),
                    mode=lax.GatherScatterMode.PROMISE_IN_BOUNDS)


def _lsr(x, n):
  return lax.shift_right_logical(x, jnp.int32(n))


def _divmod_magic(n, d, inv):
  q = (n.astype(jnp.float32) * jnp.float32(inv)).astype(jnp.int32)
  r = n - q * jnp.int32(d)
  fix_lo = r < 0
  q = jnp.where(fix_lo, q - 1, q)
  r = jnp.where(fix_lo, r + jnp.int32(d), r)
  fix_hi = r >= jnp.int32(d)
  q = jnp.where(fix_hi, q + 1, q)
  r = jnp.where(fix_hi, r - jnp.int32(d), r)
  return q, r


def _hash_chunk(pbase, iota, idx_ref, wgt_ref, outb_ref):
  for g in range(CHUNK // LANES):
    pid = (pbase + g * LANES) + iota
    hsh = pid * jnp.int32(_HASH)
    n7 = _lsr(hsh, 7)
    _, r = _divmod_magic(n7, 3125, 1.0 / 3125.0)
    tri = r * jnp.int32(128) + (hsh & jnp.int32(127))
    bsel, nidx = _divmod_magic(tri, NF, 1.0 / NF)
    m = jnp.int32(2) * _lsr(hsh, 16) + (hsh & jnp.int32(0xFFFF))
    _, r7 = _divmod_magic(m, 7, 1.0 / 7.0)
    vf = jnp.where(r7 != 0, jnp.float32(1.0), jnp.float32(0.0))
    b0 = (_lsr(hsh, 3) & jnp.int32(1023)).astype(jnp.float32) + jnp.float32(1.0)
    b1 = (_lsr(hsh, 13) & jnp.int32(1023)).astype(jnp.float32) + jnp.float32(1.0)
    b2 = (_lsr(hsh, 23) & jnp.int32(511)).astype(jnp.float32) + jnp.float32(1.0)
    inv = vf / (b0 + b1 + b2)
    sl = pl.ds(g * LANES, LANES)
    for bb in range(B):
      idx_ref[bb, sl] = jnp.where(bsel == bb, nidx, jnp.int32(-1))
    wgt_ref[0, sl] = b0 * inv
    wgt_ref[1, sl] = b1 * inv
    wgt_ref[2, sl] = b2 * inv
    outb_ref[NCH, sl] = vf


def _fma_chunk(iota, rows_ref, wgt_ref, outb_ref):
  for g in range(CHUNK // LANES):
    sl = pl.ds(g * LANES, LANES)
    w0 = wgt_ref[0, sl]
    w1 = wgt_ref[1, sl]
    w2 = wgt_ref[2, sl]
    for j in range(LANES):
      p = g * LANES + j
      r0 = rows_ref[p, 0, 0:16]
      r1 = rows_ref[p, 1, 0:16]
      r2 = rows_ref[p, 2, 0:16]
      acc = _splat(w0, j) * r0 + _splat(w1, j) * r1 + _splat(w2, j) * r2
      plsc.store_scatter(outb_ref, [iota, jnp.full((LANES,), p, jnp.int32)],
                         acc)


def _sc_body(attrs_hbm, out_hbm, idx, rows, wgt, outb, gsem, osem):
  ncores = 2
  nsub = 16
  wid = lax.axis_index("s") * ncores + lax.axis_index("c")
  rows_per_w = ROWS // (ncores * nsub)
  chunks_per_row = W // CHUNK
  nchunk = rows_per_w * chunks_per_row
  row0 = wid * rows_per_w
  iota = lax.iota(jnp.int32, LANES)

  def chunk_coords(t):
    r = row0 + t // chunks_per_row
    j0 = (t % chunks_per_row) * CHUNK
    return r, j0

  def gather_copies(s):
    return [
        pltpu.make_async_copy(
            attrs_hbm.at[bb].at[plsc.Indices(idx[s].at[bb], ignored_value=-1)],
            rows[s], gsem[s][bb])
        for bb in range(B)
    ]

  def start_gather(t, s):
    r, j0 = chunk_coords(t)
    pbase = r * W + j0
    _hash_chunk(pbase, iota, idx[s], wgt[s], outb[s])
    for c in gather_copies(s):
      c.start()

  def wait_gather(s):
    for c in gather_copies(s):
      c.wait()

  def finish_chunk(t, s):
    wait_gather(s)
    _fma_chunk(iota, rows[s], wgt[s], outb[s])
    r, j0 = chunk_coords(t)
    bb = _lsr(r, 9)
    ii = r & jnp.int32(H - 1)
    pltpu.async_copy(outb[s].at[:, 0:CHUNK],
                     out_hbm.at[bb, :, ii, pl.ds(j0, CHUNK)], osem[s])

  def wait_out(t, s):
    r, j0 = chunk_coords(t)
    bb = _lsr(r, 9)
    ii = r & jnp.int32(H - 1)
    pltpu.make_async_copy(outb[s].at[:, 0:CHUNK],
                          out_hbm.at[bb, :, ii, pl.ds(j0, CHUNK)],
                          osem[s]).wait()

  start_gather(jnp.int32(0), 0)

  @pl.loop(jnp.int32(0), jnp.int32(nchunk), step=2)
  def _(t):
    for s in (0, 1):
      tc = t + s
      tn = tc + 1

      @pl.when(tn < nchunk)
      def _():
        @pl.when(tn >= 2)
        def _():
          wait_out(tn - 2, 1 - s)
        start_gather(tn, 1 - s)

      finish_chunk(tc, s)

  wait_out(jnp.int32(nchunk - 2), 0)
  wait_out(jnp.int32(nchunk - 1), 1)


@jax.jit
def _render(attrs4):
  mesh = plsc.VectorSubcoreMesh(core_axis_name="c", subcore_axis_name="s")
  f = pl.kernel(
      _sc_body,
      out_type=jax.ShapeDtypeStruct((B, NCH + 1, H, W), jnp.float32),
      mesh=mesh,
      compiler_params=pltpu.CompilerParams(use_tc_tiling_on_sc=False,
                                           needs_layout_passes=False),
      scratch_types=dict(
          idx=[pltpu.VMEM((B, CHUNK), jnp.int32) for _ in range(2)],
          rows=[pltpu.VMEM((CHUNK, 3, LANES), jnp.float32) for _ in range(2)],
          wgt=[pltpu.VMEM((3, CHUNK), jnp.float32) for _ in range(2)],
          outb=[pltpu.VMEM((NCH + 1, CHUNK + 1), jnp.float32)
                for _ in range(2)],
          gsem=[[pltpu.SemaphoreType.DMA for _ in range(B)] for _ in range(2)],
          osem=[pltpu.SemaphoreType.DMA for _ in range(2)],
      ),
  )
  return f(attrs4)


def kernel(v, f, attrs):
  del v, f
  return _render(attrs)


# speedup vs baseline: 1.0496x; 1.0496x over previous
import functools

import jax
import jax.numpy as jnp
from jax import lax
from jax.experimental import pallas as pl
from jax.experimental.pallas import tpu as pltpu
from jax.experimental.pallas import tpu_sc as plsc

H = 512
W = 512
B = 4
NF = 100000
NTRI = B * NF
NCH = 16
CHUNK = 128
ROWS = B * H
LANES = 16

_HASH = 2654435761 - (1 << 32)

_SPLAT_DNUMS = lax.GatherDimensionNumbers(
    offset_dims=(), collapsed_slice_dims=(0,), start_index_map=(0,))


def _splat(vec, j):
  idx = jnp.full((LANES, 1), j, dtype=jnp.int32)
  return lax.gather(vec, idx, _SPLAT_DNUMS, (1,),
                    mode=lax.GatherScatterMode.PROMISE_IN_BOUNDS)


def _lsr(x, n):
  return lax.shift_right_logical(x, jnp.int32(n))


def _divmod_magic(n, d, inv):
  q = (n.astype(jnp.float32) * jnp.float32(inv)).astype(jnp.int32)
  r = n - q * jnp.int32(d)
  fix_lo = r < 0
  q = jnp.where(fix_lo, q - 1, q)
  r = jnp.where(fix_lo, r + jnp.int32(d), r)
  fix_hi = r >= jnp.int32(d)
  q = jnp.where(fix_hi, q + 1, q)
  r = jnp.where(fix_hi, r - jnp.int32(d), r)
  return q, r


def _hash_chunk(pbase, iota, idx_ref, wgt_ref, outb_ref):
  for g in range(CHUNK // LANES):
    pid = (pbase + g * LANES) + iota
    hsh = pid * jnp.int32(_HASH)
    n7 = _lsr(hsh, 7)
    _, r = _divmod_magic(n7, 3125, 1.0 / 3125.0)
    tri = r * jnp.int32(128) + (hsh & jnp.int32(127))
    m = jnp.int32(2) * _lsr(hsh, 16) + (hsh & jnp.int32(0xFFFF))
    _, r7 = _divmod_magic(m, 7, 1.0 / 7.0)
    vf = jnp.where(r7 != 0, jnp.float32(1.0), jnp.float32(0.0))
    b0 = (_lsr(hsh, 3) & jnp.int32(1023)).astype(jnp.float32) + jnp.float32(1.0)
    b1 = (_lsr(hsh, 13) & jnp.int32(1023)).astype(jnp.float32) + jnp.float32(1.0)
    b2 = (_lsr(hsh, 23) & jnp.int32(511)).astype(jnp.float32) + jnp.float32(1.0)
    inv = vf / (b0 + b1 + b2)
    sl = pl.ds(g * LANES, LANES)
    idx_ref[sl] = tri
    wgt_ref[0, sl] = b0 * inv
    wgt_ref[1, sl] = b1 * inv
    wgt_ref[2, sl] = b2 * inv
    outb_ref[NCH, sl] = vf


def _fma_chunk(iota, rows_ref, wgt_ref, outb_ref):
  for g in range(CHUNK // LANES):
    sl = pl.ds(g * LANES, LANES)
    w0 = wgt_ref[0, sl]
    w1 = wgt_ref[1, sl]
    w2 = wgt_ref[2, sl]
    for j in range(LANES):
      p = g * LANES + j
      r0 = rows_ref[p, 0, 0:16]
      r1 = rows_ref[p, 1, 0:16]
      r2 = rows_ref[p, 2, 0:16]
      acc = _splat(w0, j) * r0 + _splat(w1, j) * r1 + _splat(w2, j) * r2
      plsc.store_scatter(outb_ref, [iota, jnp.full((LANES,), p, jnp.int32)],
                         acc)


def _sc_body(attrs_hbm, out_hbm, idx, rows, wgt, outb, gsem, osem):
  ncores = 2
  nsub = 16
  wid = lax.axis_index("s") * ncores + lax.axis_index("c")
  rows_per_w = ROWS // (ncores * nsub)
  chunks_per_row = W // CHUNK
  nchunk = rows_per_w * chunks_per_row
  row0 = wid * rows_per_w
  iota = lax.iota(jnp.int32, LANES)

  def chunk_coords(t):
    r = row0 + t // chunks_per_row
    j0 = (t % chunks_per_row) * CHUNK
    return r, j0

  def gather_copy(s):
    return pltpu.make_async_copy(attrs_hbm.at[0].at[idx[s]], rows[s],
                                 gsem[s])

  def start_gather(t, s):
    r, j0 = chunk_coords(t)
    pbase = r * W + j0
    _hash_chunk(pbase, iota, idx[s], wgt[s], outb[s])
    gather_copy(s).start()

  def wait_gather(s):
    gather_copy(s).wait()

  def finish_chunk(t, s):
    wait_gather(s)
    _fma_chunk(iota, rows[s], wgt[s], outb[s])
    r, j0 = chunk_coords(t)
    bb = _lsr(r, 9)
    ii = r & jnp.int32(H - 1)
    pltpu.async_copy(outb[s].at[:, 0:CHUNK],
                     out_hbm.at[bb, :, ii, pl.ds(j0, CHUNK)], osem[s])

  def wait_out(t, s):
    r, j0 = chunk_coords(t)
    bb = _lsr(r, 9)
    ii = r & jnp.int32(H - 1)
    pltpu.make_async_copy(outb[s].at[:, 0:CHUNK],
                          out_hbm.at[bb, :, ii, pl.ds(j0, CHUNK)],
                          osem[s]).wait()

  start_gather(jnp.int32(0), 0)

  @pl.loop(jnp.int32(0), jnp.int32(nchunk), step=2)
  def _(t):
    for s in (0, 1):
      tc = t + s
      tn = tc + 1

      @pl.when(tn < nchunk)
      def _():
        @pl.when(tn >= 2)
        def _():
          wait_out(tn - 2, 1 - s)
        start_gather(tn, 1 - s)

      finish_chunk(tc, s)

  wait_out(jnp.int32(nchunk - 2), 0)
  wait_out(jnp.int32(nchunk - 1), 1)


@jax.jit
def _render(attrs4):
  mesh = plsc.VectorSubcoreMesh(core_axis_name="c", subcore_axis_name="s")
  f = pl.kernel(
      _sc_body,
      out_type=jax.ShapeDtypeStruct((B, NCH + 1, H, W), jnp.float32),
      mesh=mesh,
      compiler_params=pltpu.CompilerParams(use_tc_tiling_on_sc=False,
                                           needs_layout_passes=False),
      scratch_types=dict(
          idx=[pltpu.VMEM((CHUNK,), jnp.int32) for _ in range(2)],
          rows=[pltpu.VMEM((CHUNK, 3, LANES), jnp.float32) for _ in range(2)],
          wgt=[pltpu.VMEM((3, CHUNK), jnp.float32) for _ in range(2)],
          outb=[pltpu.VMEM((NCH + 1, CHUNK + 1), jnp.float32)
                for _ in range(2)],
          gsem=[pltpu.SemaphoreType.DMA for _ in range(2)],
          osem=[pltpu.SemaphoreType.DMA for _ in range(2)],
      ),
  )
  return f(attrs4)


def kernel(v, f, attrs):
  del v, f
  return _render(attrs)


# speedup vs baseline: 1.8942x; 1.8048x over previous
import functools

import jax
import jax.numpy as jnp
from jax import lax
from jax.experimental import pallas as pl
from jax.experimental.pallas import tpu as pltpu
from jax.experimental.pallas import tpu_sc as plsc

H = 512
W = 512
B = 4
NF = 100000
NTRI = B * NF
NCH = 16
CHUNK = 128
ROWS = B * H
LANES = 16

_HASH = 2654435761 - (1 << 32)

_SPLAT_DNUMS = lax.GatherDimensionNumbers(
    offset_dims=(), collapsed_slice_dims=(0,), start_index_map=(0,))


def _splat(vec, j):
  idx = jnp.full((LANES, 1), j, dtype=jnp.int32)
  return lax.gather(vec, idx, _SPLAT_DNUMS, (1,),
                    mode=lax.GatherScatterMode.PROMISE_IN_BOUNDS)


def _lsr(x, n):
  return lax.shift_right_logical(x, jnp.int32(n))


def _divmod_magic(n, d, inv):
  q = (n.astype(jnp.float32) * jnp.float32(inv)).astype(jnp.int32)
  r = n - q * jnp.int32(d)
  fix_lo = r < 0
  q = jnp.where(fix_lo, q - 1, q)
  r = jnp.where(fix_lo, r + jnp.int32(d), r)
  fix_hi = r >= jnp.int32(d)
  q = jnp.where(fix_hi, q + 1, q)
  r = jnp.where(fix_hi, r - jnp.int32(d), r)
  return q, r


def _hash_chunk(pbase, iota, idx_ref, wgt_ref, outb_ref):
  for g in range(CHUNK // LANES):
    pid = (pbase + g * LANES) + iota
    hsh = pid * jnp.int32(_HASH)
    n7 = _lsr(hsh, 7)
    _, r = _divmod_magic(n7, 3125, 1.0 / 3125.0)
    tri = r * jnp.int32(128) + (hsh & jnp.int32(127))
    m = jnp.int32(2) * _lsr(hsh, 16) + (hsh & jnp.int32(0xFFFF))
    _, r7 = _divmod_magic(m, 7, 1.0 / 7.0)
    vf = jnp.where(r7 != 0, jnp.float32(1.0), jnp.float32(0.0))
    b0 = (_lsr(hsh, 3) & jnp.int32(1023)).astype(jnp.float32) + jnp.float32(1.0)
    b1 = (_lsr(hsh, 13) & jnp.int32(1023)).astype(jnp.float32) + jnp.float32(1.0)
    b2 = (_lsr(hsh, 23) & jnp.int32(511)).astype(jnp.float32) + jnp.float32(1.0)
    inv = vf / (b0 + b1 + b2)
    sl = pl.ds(g * LANES, LANES)
    idx_ref[sl] = tri
    wgt_ref[0, sl] = b0 * inv
    wgt_ref[1, sl] = b1 * inv
    wgt_ref[2, sl] = b2 * inv
    outb_ref[NCH, sl] = vf


def _fma_chunk(iota, rows_ref, wgt_ref, outb_ref):
  for g in range(CHUNK // LANES):
    sl = pl.ds(g * LANES, LANES)
    w0 = wgt_ref[0, sl]
    w1 = wgt_ref[1, sl]
    w2 = wgt_ref[2, sl]
    accs = []
    for j in range(LANES):
      p = g * LANES + j
      r0 = rows_ref[p, 0:16]
      r1 = rows_ref[p, 16:32]
      r2 = rows_ref[p, 32:48]
      accs.append(_splat(w0, j) * r0 + _splat(w1, j) * r1
                  + _splat(w2, j) * r2)
    for j in range(LANES):
      p = g * LANES + j
      plsc.store_scatter(outb_ref, [iota, jnp.full((LANES,), p, jnp.int32)],
                         accs[j])


def _sc_body(attrs_hbm, out_hbm, idx, rows, wgt, outb, gsem, osem):
  ncores = 2
  nsub = 16
  wid = lax.axis_index("s") * ncores + lax.axis_index("c")
  rows_per_w = ROWS // (ncores * nsub)
  chunks_per_row = W // CHUNK
  nchunk = rows_per_w * chunks_per_row
  row0 = wid * rows_per_w
  iota = lax.iota(jnp.int32, LANES)

  def chunk_coords(t):
    r = row0 + t // chunks_per_row
    j0 = (t % chunks_per_row) * CHUNK
    return r, j0

  def gather_copy(s):
    return pltpu.make_async_copy(attrs_hbm.at[idx[s]], rows[s], gsem[s])

  def start_gather(t, s):
    r, j0 = chunk_coords(t)
    pbase = r * W + j0
    _hash_chunk(pbase, iota, idx[s], wgt[s], outb[s])
    gather_copy(s).start()

  def wait_gather(s):
    gather_copy(s).wait()

  def finish_chunk(t, s):
    wait_gather(s)
    _fma_chunk(iota, rows[s], wgt[s], outb[s])
    r, j0 = chunk_coords(t)
    bb = _lsr(r, 9)
    ii = r & jnp.int32(H - 1)
    pltpu.async_copy(outb[s].at[:, 0:CHUNK],
                     out_hbm.at[bb, :, ii, pl.ds(j0, CHUNK)], osem[s])

  def wait_out(t, s):
    r, j0 = chunk_coords(t)
    bb = _lsr(r, 9)
    ii = r & jnp.int32(H - 1)
    pltpu.make_async_copy(outb[s].at[:, 0:CHUNK],
                          out_hbm.at[bb, :, ii, pl.ds(j0, CHUNK)],
                          osem[s]).wait()

  start_gather(jnp.int32(0), 0)

  @pl.loop(jnp.int32(0), jnp.int32(nchunk), step=2)
  def _(t):
    for s in (0, 1):
      tc = t + s
      tn = tc + 1

      @pl.when(tn < nchunk)
      def _():
        @pl.when(tn >= 2)
        def _():
          wait_out(tn - 2, 1 - s)
        start_gather(tn, 1 - s)

      finish_chunk(tc, s)

  wait_out(jnp.int32(nchunk - 2), 0)
  wait_out(jnp.int32(nchunk - 1), 1)


@jax.jit
def _render(attrs2):
  mesh = plsc.VectorSubcoreMesh(core_axis_name="c", subcore_axis_name="s")
  f = pl.kernel(
      _sc_body,
      out_type=jax.ShapeDtypeStruct((B, NCH + 1, H, W), jnp.float32),
      mesh=mesh,
      compiler_params=pltpu.CompilerParams(use_tc_tiling_on_sc=False,
                                           needs_layout_passes=False),
      scratch_types=dict(
          idx=[pltpu.VMEM((CHUNK,), jnp.int32) for _ in range(2)],
          rows=[pltpu.VMEM((CHUNK, 48), jnp.float32) for _ in range(2)],
          wgt=[pltpu.VMEM((3, CHUNK), jnp.float32) for _ in range(2)],
          outb=[pltpu.VMEM((NCH + 1, CHUNK + 1), jnp.float32)
                for _ in range(2)],
          gsem=[pltpu.SemaphoreType.DMA for _ in range(2)],
          osem=[pltpu.SemaphoreType.DMA for _ in range(2)],
      ),
  )
  return f(attrs2)


def kernel(v, f, attrs):
  del v, f
  return _render(attrs.reshape(NTRI, 48))
